# baseline (speedup 1.0000x reference)
import functools

import jax
import jax.numpy as jnp
from jax import lax
from jax.experimental import pallas as pl
from jax.experimental.pallas import tpu as pltpu
from jax.experimental.pallas import tpu_sc as plsc

B = 16
C = 384
CH = C // 2
HW = 4096
P = 256
NC, NS = 2, 16
NW = NC * NS
RPT = B * P // NW
L = 16
NV = CH // L
OPAD = 2 * 128


def _rsqrt(s):
    i = plsc.bitcast(s, jnp.int32)
    i = jnp.int32(0x5F3759DF) - lax.shift_right_arithmetic(i, 1)
    y = plsc.bitcast(i, jnp.float32)
    half = s * 0.5
    for _ in range(3):
        y = y * (1.5 - half * y * y)
    return y


def _sc_body(x_hbm, pid_hbm, out_hbm, pid_v, idx0, idx1, idx2, idx3,
             rows_v, ssq_all, out_local, sem, sem1, sem2, sem3, semw):
    cid = lax.axis_index("c")
    sid = lax.axis_index("s")
    wid = cid * NS + sid
    b = lax.div(wid, 2)
    poff = lax.rem(wid, 2) * RPT

    RQ = RPT // 4

    with jax.named_scope("idx_setup"):
        pltpu.sync_copy(pid_hbm.at[pl.ds(poff, RPT)], pid_v)

        base = b * HW
        idxq = (idx0, idx1, idx2, idx3)
        for q in range(4):
            for k in range(RQ // L):
                idxq[q][pl.ds(k * L, L)] = \
                    pid_v[pl.ds(q * RQ + k * L, L)] + base

    with jax.named_scope("row_gather_start"):
        sems = (sem, sem1, sem2, sem3)
        dq = [
            pltpu.async_copy(x_hbm.at[idxq[q]],
                             rows_v.at[pl.ds(q * RQ, RQ)], sems[q])
            for q in range(4)
        ]

    lane15 = jnp.full((L,), L - 1, dtype=jnp.int32)

    def row_norm(i, base_r):
        rs = [base_r + i * 2, base_r + i * 2 + 1]
        vs, facs = [], []
        for r in rs:
            v = [rows_v[r, pl.ds(t * L, L)] for t in range(NV)]
            acc = v[0] * v[0]
            for t in range(1, NV):
                acc = acc + v[t] * v[t]
            ssq_all[r] = plsc.cumsum(acc)
            vs.append(v)
        for r in rs:
            rv = jnp.full((L,), r, dtype=jnp.int32)
            s = plsc.load_gather(ssq_all, [rv, lane15])
            norm = s * _rsqrt(s)
            facs.append(1.0 / (norm + 1e-7))
        for r, v, fac in zip(rs, vs, facs):
            for t in range(NV):
                out_local[r, pl.ds(t * L, L)] = v[t] * fac
        return base_r

    with jax.named_scope("half0"):
        dq[0].wait()
        lax.fori_loop(0, RQ // 2, row_norm, 0)
        dq[1].wait()
        lax.fori_loop(0, RQ // 2, row_norm, RQ)
        w0 = pltpu.async_copy(
            out_local.at[pl.ds(0, 2 * RQ)],
            out_hbm.at[pl.ds(wid * RPT, 2 * RQ)], semw)

    with jax.named_scope("half1"):
        dq[2].wait()
        lax.fori_loop(0, RQ // 2, row_norm, 2 * RQ)
        dq[3].wait()
        lax.fori_loop(0, RQ // 2, row_norm, 3 * RQ)
        pltpu.sync_copy(out_local.at[pl.ds(2 * RQ, 2 * RQ)],
                        out_hbm.at[pl.ds(wid * RPT + 2 * RQ, 2 * RQ)])
        w0.wait()


@jax.jit
def _run(x4, patch_ids):
    xt = jnp.transpose(x4, (0, 2, 3, 1)).reshape(B * HW, C)
    mesh = plsc.VectorSubcoreMesh(
        core_axis_name="c", subcore_axis_name="s",
        num_cores=NC, num_subcores=NS)
    f = pl.kernel(
        _sc_body,
        out_type=jax.ShapeDtypeStruct((B * P, OPAD), jnp.float32),
        mesh=mesh,
        scratch_types=[
            pltpu.VMEM((RPT,), jnp.int32),
            pltpu.VMEM((RPT // 4,), jnp.int32),
            pltpu.VMEM((RPT // 4,), jnp.int32),
            pltpu.VMEM((RPT // 4,), jnp.int32),
            pltpu.VMEM((RPT // 4,), jnp.int32),
            pltpu.VMEM((RPT, C), jnp.float32),
            pltpu.VMEM((RPT, L), jnp.float32),
            pltpu.VMEM((RPT, OPAD), jnp.float32),
            pltpu.SemaphoreType.DMA,
            pltpu.SemaphoreType.DMA,
            pltpu.SemaphoreType.DMA,
            pltpu.SemaphoreType.DMA,
            pltpu.SemaphoreType.DMA,
        ],
        compiler_params=pltpu.CompilerParams(
            use_tc_tiling_on_sc=True, needs_layout_passes=False),
    )
    return f(xt, patch_ids)[:, :CH]


def kernel(x, num_patches, patch_ids):
    out = _run(x, patch_ids)
    return (out, patch_ids)


# speedup vs baseline: 1.0336x; 1.0336x over previous
import functools

import jax
import jax.numpy as jnp
from jax import lax
from jax.experimental import pallas as pl
from jax.experimental.pallas import tpu as pltpu
from jax.experimental.pallas import tpu_sc as plsc

B = 16
C = 384
CH = C // 2
HW = 4096
P = 256
NC, NS = 2, 16
NW = NC * NS
RPT = B * P // NW
L = 16
NV = CH // L
OPAD = 2 * 128


def _rsqrt(s):
    i = plsc.bitcast(s, jnp.int32)
    i = jnp.int32(0x5F3759DF) - lax.shift_right_arithmetic(i, 1)
    y = plsc.bitcast(i, jnp.float32)
    half = s * 0.5
    for _ in range(3):
        y = y * (1.5 - half * y * y)
    return y


def _sc_body(x_hbm, pid_hbm, out_hbm, pid_v, idx0, idx1, rows_v,
             ssq_all, out_local, sem, sem1, semw):
    cid = lax.axis_index("c")
    sid = lax.axis_index("s")
    wid = cid * NS + sid
    b = lax.div(wid, 2)
    poff = lax.rem(wid, 2) * RPT

    RH = RPT // 2

    with jax.named_scope("idx_setup"):
        pltpu.sync_copy(pid_hbm.at[pl.ds(poff, RPT)], pid_v)

        base = b * HW
        for k in range(RH // L):
            idx0[pl.ds(k * L, L)] = pid_v[pl.ds(k * L, L)] + base
            idx1[pl.ds(k * L, L)] = pid_v[pl.ds(RH + k * L, L)] + base

    with jax.named_scope("row_gather_start"):
        d0 = pltpu.async_copy(x_hbm.at[idx0], rows_v.at[pl.ds(0, RH)], sem)
        d1 = pltpu.async_copy(x_hbm.at[idx1], rows_v.at[pl.ds(RH, RH)],
                              sem1)

    lane15 = jnp.full((L,), L - 1, dtype=jnp.int32)

    def row_norm(i, base_r):
        rs = [base_r + i * 2, base_r + i * 2 + 1]
        vs, facs = [], []
        for r in rs:
            v = [rows_v[r, pl.ds(t * L, L)] for t in range(NV)]
            acc = v[0] * v[0]
            for t in range(1, NV):
                acc = acc + v[t] * v[t]
            ssq_all[r] = plsc.cumsum(acc)
            vs.append(v)
        for r in rs:
            rv = jnp.full((L,), r, dtype=jnp.int32)
            s = plsc.load_gather(ssq_all, [rv, lane15])
            norm = s * _rsqrt(s)
            facs.append(1.0 / (norm + 1e-7))
        for r, v, fac in zip(rs, vs, facs):
            for t in range(NV):
                out_local[r, pl.ds(t * L, L)] = v[t] * fac
        return base_r

    with jax.named_scope("half0"):
        d0.wait()
        lax.fori_loop(0, RH // 2, row_norm, 0)
        w0 = pltpu.async_copy(
            out_local.at[pl.ds(0, RH)],
            out_hbm.at[pl.ds(wid * RPT, RH)], semw)

    with jax.named_scope("half1"):
        d1.wait()
        lax.fori_loop(0, RH // 2, row_norm, RH)
        pltpu.sync_copy(out_local.at[pl.ds(RH, RH)],
                        out_hbm.at[pl.ds(wid * RPT + RH, RH)])
        w0.wait()


@jax.jit
def _run(x4, patch_ids):
    xt = jnp.transpose(x4, (0, 2, 3, 1)).reshape(B * HW, C)
    mesh = plsc.VectorSubcoreMesh(
        core_axis_name="c", subcore_axis_name="s",
        num_cores=NC, num_subcores=NS)
    f = pl.kernel(
        _sc_body,
        out_type=jax.ShapeDtypeStruct((B * P, OPAD), jnp.float32),
        mesh=mesh,
        scratch_types=[
            pltpu.VMEM((RPT,), jnp.int32),
            pltpu.VMEM((RPT // 2,), jnp.int32),
            pltpu.VMEM((RPT // 2,), jnp.int32),
            pltpu.VMEM((RPT, C), jnp.float32),
            pltpu.VMEM((RPT, L), jnp.float32),
            pltpu.VMEM((RPT, OPAD), jnp.float32),
            pltpu.SemaphoreType.DMA,
            pltpu.SemaphoreType.DMA,
            pltpu.SemaphoreType.DMA,
        ],
        compiler_params=pltpu.CompilerParams(
            use_tc_tiling_on_sc=True, needs_layout_passes=False),
    )
    return f(xt, patch_ids)[:, :CH]


def kernel(x, num_patches, patch_ids):
    out = _run(x, patch_ids)
    return (out, patch_ids)


# speedup vs baseline: 1.0690x; 1.0343x over previous
import functools

import jax
import jax.numpy as jnp
from jax import lax
from jax.experimental import pallas as pl
from jax.experimental.pallas import tpu as pltpu
from jax.experimental.pallas import tpu_sc as plsc

B = 16
C = 384
CH = C // 2
HW = 4096
P = 256
NC, NS = 2, 16
NW = NC * NS
RPT = B * P // NW
L = 16
NV = CH // L
OPAD = 2 * 128


def _rsqrt(s):
    i = plsc.bitcast(s, jnp.int32)
    i = jnp.int32(0x5F3759DF) - lax.shift_right_arithmetic(i, 1)
    y = plsc.bitcast(i, jnp.float32)
    half = s * 0.5
    for _ in range(2):
        y = y * (1.5 - half * y * y)
    return y


def _sc_body(x_hbm, pid_hbm, out_hbm, pid_v, idx0, idx1, rows_v,
             ssq_all, out_local, sem, sem1, semw):
    cid = lax.axis_index("c")
    sid = lax.axis_index("s")
    wid = cid * NS + sid
    b = lax.div(wid, 2)
    poff = lax.rem(wid, 2) * RPT

    RH = RPT // 2

    with jax.named_scope("idx_setup"):
        pltpu.sync_copy(pid_hbm.at[pl.ds(poff, RPT)], pid_v)

        base = b * HW
        for k in range(RH // L):
            idx0[pl.ds(k * L, L)] = pid_v[pl.ds(k * L, L)] + base
            idx1[pl.ds(k * L, L)] = pid_v[pl.ds(RH + k * L, L)] + base

    with jax.named_scope("row_gather_start"):
        d0 = pltpu.async_copy(x_hbm.at[idx0], rows_v.at[pl.ds(0, RH)], sem)
        d1 = pltpu.async_copy(x_hbm.at[idx1], rows_v.at[pl.ds(RH, RH)],
                              sem1)

    lane15 = jnp.full((L,), L - 1, dtype=jnp.int32)

    def row_norm(i, base_r):
        rs = [base_r + i * 4 + u for u in range(4)]
        vs, facs = [], []
        for r in rs:
            v = [rows_v[r, pl.ds(t * L, L)] for t in range(NV)]
            acc = v[0] * v[0]
            for t in range(1, NV):
                acc = acc + v[t] * v[t]
            ssq_all[r] = plsc.cumsum(acc)
            vs.append(v)
        for r in rs:
            rv = jnp.full((L,), r, dtype=jnp.int32)
            s = plsc.load_gather(ssq_all, [rv, lane15])
            norm = s * _rsqrt(s)
            facs.append(1.0 / (norm + 1e-7))
        for r, v, fac in zip(rs, vs, facs):
            for t in range(NV):
                out_local[r, pl.ds(t * L, L)] = v[t] * fac
        return base_r

    with jax.named_scope("half0"):
        d0.wait()
        lax.fori_loop(0, RH // 4, row_norm, 0)
        w0 = pltpu.async_copy(
            out_local.at[pl.ds(0, RH)],
            out_hbm.at[pl.ds(wid * RPT, RH)], semw)

    with jax.named_scope("half1"):
        d1.wait()
        lax.fori_loop(0, RH // 4, row_norm, RH)
        pltpu.sync_copy(out_local.at[pl.ds(RH, RH)],
                        out_hbm.at[pl.ds(wid * RPT + RH, RH)])
        w0.wait()


@jax.jit
def _run(x4, patch_ids):
    xt = jnp.transpose(x4, (0, 2, 3, 1)).reshape(B * HW, C)
    mesh = plsc.VectorSubcoreMesh(
        core_axis_name="c", subcore_axis_name="s",
        num_cores=NC, num_subcores=NS)
    f = pl.kernel(
        _sc_body,
        out_type=jax.ShapeDtypeStruct((B * P, OPAD), jnp.float32),
        mesh=mesh,
        scratch_types=[
            pltpu.VMEM((RPT,), jnp.int32),
            pltpu.VMEM((RPT // 2,), jnp.int32),
            pltpu.VMEM((RPT // 2,), jnp.int32),
            pltpu.VMEM((RPT, C), jnp.float32),
            pltpu.VMEM((RPT, L), jnp.float32),
            pltpu.VMEM((RPT, OPAD), jnp.float32),
            pltpu.SemaphoreType.DMA,
            pltpu.SemaphoreType.DMA,
            pltpu.SemaphoreType.DMA,
        ],
        compiler_params=pltpu.CompilerParams(
            use_tc_tiling_on_sc=True, needs_layout_passes=False),
    )
    return f(xt, patch_ids)[:, :CH]


def kernel(x, num_patches, patch_ids):
    out = _run(x, patch_ids)
    return (out, patch_ids)
